# Initial kernel scaffold; baseline (speedup 1.0000x reference)
import functools

import jax
import jax.numpy as jnp
from jax import lax
from jax.experimental import pallas as pl
from jax.experimental.pallas import tpu as pltpu
from jax.experimental.pallas import tpu_sc as plsc

N_NODES = 10000
D = 128
N_EDGES = 320000
N_GRAPHS = 64
NUM_LAYERS = 3

NC = 2
NS = 16
NW = NC * NS
E_PER_W = N_EDGES // NW
CHUNK = 80
STEPS = E_PER_W // CHUNK
ROWS_PER_TILE = N_NODES // NS

ROW_BLK = 1000
GRID = N_NODES // ROW_BLK


_sc_mesh = plsc.VectorSubcoreMesh(core_axis_name="c", subcore_axis_name="s")


@functools.partial(
    pl.kernel,
    out_type=jax.ShapeDtypeStruct((NC, N_NODES, D), jnp.float32),
    mesh=_sc_mesh,
    scratch_types=[
        pltpu.VMEM((CHUNK,), jnp.int32),
        pltpu.VMEM((CHUNK,), jnp.int32),
        pltpu.VMEM((CHUNK, D), jnp.float32),
        pltpu.VMEM((ROWS_PER_TILE, D), jnp.float32),
        pltpu.VMEM_SHARED((N_NODES, D), jnp.float32),
        pltpu.SemaphoreType.DMA,
    ],
)
def _sc_aggregate(m_hbm, src_hbm, dst_hbm, zeros_hbm, out_hbm,
                  src_v, dst_v, rows_v, stage_v, acc_sh, sem):
    c = lax.axis_index("c")
    s = lax.axis_index("s")
    wid = s * NC + c
    r0 = s * ROWS_PER_TILE

    pltpu.sync_copy(zeros_hbm.at[pl.ds(r0, ROWS_PER_TILE)], stage_v)
    pltpu.sync_copy(stage_v, acc_sh.at[pl.ds(r0, ROWS_PER_TILE)])
    plsc.subcore_barrier()

    ebase = wid * E_PER_W

    def body(step, carry):
        off = ebase + step * CHUNK
        pltpu.sync_copy(src_hbm.at[pl.ds(off, CHUNK)], src_v)
        pltpu.sync_copy(dst_hbm.at[pl.ds(off, CHUNK)], dst_v)
        pltpu.async_copy(m_hbm.at[src_v], rows_v, sem).wait()
        pltpu.sync_copy(rows_v, acc_sh.at[dst_v], add=True)
        return carry

    lax.fori_loop(0, STEPS, body, 0)
    plsc.subcore_barrier()

    pltpu.sync_copy(acc_sh.at[pl.ds(r0, ROWS_PER_TILE)], stage_v)
    pltpu.sync_copy(stage_v, out_hbm.at[c, pl.ds(r0, ROWS_PER_TILE)])


def _mm_body(x_ref, w_ref, o_ref):
    o_ref[...] = jnp.dot(x_ref[...], w_ref[...],
                         preferred_element_type=jnp.float32)


_mm = pl.pallas_call(
    _mm_body,
    grid=(GRID,),
    in_specs=[
        pl.BlockSpec((ROW_BLK, D), lambda i: (i, 0)),
        pl.BlockSpec((D, D), lambda i: (0, 0)),
    ],
    out_specs=pl.BlockSpec((ROW_BLK, D), lambda i: (i, 0)),
    out_shape=jax.ShapeDtypeStruct((N_NODES, D), jnp.float32),
)


def _gru(part_ref, h_ref, wih_ref, whh_ref, bih_ref, bhh_ref):
    agg = part_ref[0] + part_ref[1]
    h = h_ref[...]
    gi = jnp.dot(agg, wih_ref[...], preferred_element_type=jnp.float32)
    gi = gi + bih_ref[...]
    gh = jnp.dot(h, whh_ref[...], preferred_element_type=jnp.float32)
    gh = gh + bhh_ref[...]
    r = jax.nn.sigmoid(gi[:, :D] + gh[:, :D])
    z = jax.nn.sigmoid(gi[:, D:2 * D] + gh[:, D:2 * D])
    n = jnp.tanh(gi[:, 2 * D:] + r * gh[:, 2 * D:])
    return (1.0 - z) * n + z * h


def _gru_mm_body(part_ref, h_ref, wih_ref, whh_ref, bih_ref, bhh_ref,
                 wnext_ref, h_out_ref, m_out_ref):
    hn = _gru(part_ref, h_ref, wih_ref, whh_ref, bih_ref, bhh_ref)
    h_out_ref[...] = hn
    m_out_ref[...] = jnp.dot(hn, wnext_ref[...],
                             preferred_element_type=jnp.float32)


_gru_mm = pl.pallas_call(
    _gru_mm_body,
    grid=(GRID,),
    in_specs=[
        pl.BlockSpec((NC, ROW_BLK, D), lambda i: (0, i, 0)),
        pl.BlockSpec((ROW_BLK, D), lambda i: (i, 0)),
        pl.BlockSpec((D, 3 * D), lambda i: (0, 0)),
        pl.BlockSpec((D, 3 * D), lambda i: (0, 0)),
        pl.BlockSpec((1, 3 * D), lambda i: (0, 0)),
        pl.BlockSpec((1, 3 * D), lambda i: (0, 0)),
        pl.BlockSpec((D, D), lambda i: (0, 0)),
    ],
    out_specs=[
        pl.BlockSpec((ROW_BLK, D), lambda i: (i, 0)),
        pl.BlockSpec((ROW_BLK, D), lambda i: (i, 0)),
    ],
    out_shape=[
        jax.ShapeDtypeStruct((N_NODES, D), jnp.float32),
        jax.ShapeDtypeStruct((N_NODES, D), jnp.float32),
    ],
)


def _gru_pool_body(part_ref, h_ref, wih_ref, whh_ref, bih_ref, bhh_ref,
                   bb_ref, o_ref):
    hn = _gru(part_ref, h_ref, wih_ref, whh_ref, bih_ref, bhh_ref)
    onehot = (bb_ref[...] ==
              lax.broadcasted_iota(jnp.float32, (ROW_BLK, D), 1))
    contrib = lax.dot_general(onehot.astype(jnp.float32), hn,
                              (((0,), (0,)), ((), ())),
                              preferred_element_type=jnp.float32)

    @pl.when(pl.program_id(0) == 0)
    def _():
        o_ref[...] = jnp.zeros_like(o_ref)

    o_ref[...] += contrib


_gru_pool = pl.pallas_call(
    _gru_pool_body,
    grid=(GRID,),
    in_specs=[
        pl.BlockSpec((NC, ROW_BLK, D), lambda i: (0, i, 0)),
        pl.BlockSpec((ROW_BLK, D), lambda i: (i, 0)),
        pl.BlockSpec((D, 3 * D), lambda i: (0, 0)),
        pl.BlockSpec((D, 3 * D), lambda i: (0, 0)),
        pl.BlockSpec((1, 3 * D), lambda i: (0, 0)),
        pl.BlockSpec((1, 3 * D), lambda i: (0, 0)),
        pl.BlockSpec((ROW_BLK, D), lambda i: (i, 0)),
    ],
    out_specs=pl.BlockSpec((D, D), lambda i: (0, 0)),
    out_shape=jax.ShapeDtypeStruct((D, D), jnp.float32),
)


def kernel(x, edge_index, batch, weight, w_ih, w_hh, b_ih, b_hh):
    src = edge_index[0].astype(jnp.int32)
    dst = edge_index[1].astype(jnp.int32)
    wih_t = w_ih.T
    whh_t = w_hh.T
    bih = b_ih.reshape(1, 3 * D)
    bhh = b_hh.reshape(1, 3 * D)
    zeros = jnp.zeros((N_NODES, D), jnp.float32)
    batch_b = jnp.broadcast_to(
        batch.astype(jnp.float32)[:, None], (N_NODES, D))

    h = x
    m = _mm(x, weight[0])
    for i in range(NUM_LAYERS - 1):
        part = _sc_aggregate(m, src, dst, zeros)
        h, m = _gru_mm(part, h, wih_t, whh_t, bih, bhh, weight[i + 1])
    part = _sc_aggregate(m, src, dst, zeros)
    out_pad = _gru_pool(part, h, wih_t, whh_t, bih, bhh, batch_b)
    return out_pad[:N_GRAPHS]


# speedup vs baseline: 4.9649x; 4.9649x over previous
import functools

import jax
import jax.numpy as jnp
from jax import lax
from jax.experimental import pallas as pl
from jax.experimental.pallas import tpu as pltpu
from jax.experimental.pallas import tpu_sc as plsc

N_NODES = 10000
D = 128
N_EDGES = 320000
N_GRAPHS = 64
NUM_LAYERS = 3

NC = 2
NS = 16
NW = NC * NS
E_PER_W = N_EDGES // NW
CHUNK = 80
STEPS = E_PER_W // CHUNK
N_PAD = 10240
ROWS_PER_TILE = N_PAD // NS

ROW_BLK = 1000
GRID = N_NODES // ROW_BLK


@functools.cache
def _make_sc_aggregate():
    mesh = plsc.VectorSubcoreMesh(core_axis_name="c", subcore_axis_name="s",
                                  num_cores=NC)

    @functools.partial(
        pl.kernel,
        out_type=jax.ShapeDtypeStruct((NC, N_PAD, D), jnp.float32),
        mesh=mesh,
        scratch_types=[
            pltpu.VMEM((CHUNK,), jnp.int32),
            pltpu.VMEM((CHUNK,), jnp.int32),
            pltpu.VMEM((CHUNK, D), jnp.float32),
            pltpu.VMEM_SHARED((N_PAD, D), jnp.float32),
            pltpu.SemaphoreType.DMA,
        ],
    )
    def sc_aggregate(m_hbm, src_hbm, dst_hbm, zeros_hbm, out_hbm,
                     src_v, dst_v, rows_v, acc_sh, sem):
        c = lax.axis_index("c")
        s = lax.axis_index("s")
        wid = s * NC + c
        r0 = s * ROWS_PER_TILE

        pltpu.sync_copy(zeros_hbm.at[pl.ds(0, CHUNK)], rows_v)
        for k in range(ROWS_PER_TILE // CHUNK):
            pltpu.sync_copy(rows_v, acc_sh.at[pl.ds(r0 + k * CHUNK, CHUNK)])
        plsc.subcore_barrier()

        ebase = wid * E_PER_W

        def body(step, carry):
            off = ebase + step * CHUNK
            pltpu.sync_copy(src_hbm.at[pl.ds(off, CHUNK)], src_v)
            pltpu.sync_copy(dst_hbm.at[pl.ds(off, CHUNK)], dst_v)
            pltpu.async_copy(m_hbm.at[src_v], rows_v, sem).wait()
            pltpu.sync_copy(rows_v, acc_sh.at[dst_v], add=True)
            return carry

        lax.fori_loop(0, STEPS, body, 0)
        plsc.subcore_barrier()

        for k in range(ROWS_PER_TILE // CHUNK):
            pltpu.sync_copy(acc_sh.at[pl.ds(r0 + k * CHUNK, CHUNK)], rows_v)
            pltpu.sync_copy(rows_v, out_hbm.at[c, pl.ds(r0 + k * CHUNK, CHUNK)])

    return sc_aggregate


def _mm_body(x_ref, w_ref, o_ref):
    o_ref[...] = jnp.dot(x_ref[...], w_ref[...],
                         preferred_element_type=jnp.float32)


_mm = pl.pallas_call(
    _mm_body,
    grid=(GRID,),
    in_specs=[
        pl.BlockSpec((ROW_BLK, D), lambda i: (i, 0)),
        pl.BlockSpec((D, D), lambda i: (0, 0)),
    ],
    out_specs=pl.BlockSpec((ROW_BLK, D), lambda i: (i, 0)),
    out_shape=jax.ShapeDtypeStruct((N_NODES, D), jnp.float32),
)


def _gru(part_ref, h_ref, wih_ref, whh_ref, bih_ref, bhh_ref):
    agg = part_ref[0] + part_ref[1]
    h = h_ref[...]
    gi = jnp.dot(agg, wih_ref[...], preferred_element_type=jnp.float32)
    gi = gi + bih_ref[...]
    gh = jnp.dot(h, whh_ref[...], preferred_element_type=jnp.float32)
    gh = gh + bhh_ref[...]
    r = jax.nn.sigmoid(gi[:, :D] + gh[:, :D])
    z = jax.nn.sigmoid(gi[:, D:2 * D] + gh[:, D:2 * D])
    n = jnp.tanh(gi[:, 2 * D:] + r * gh[:, 2 * D:])
    return (1.0 - z) * n + z * h


def _gru_mm_body(part_ref, h_ref, wih_ref, whh_ref, bih_ref, bhh_ref,
                 wnext_ref, h_out_ref, m_out_ref):
    hn = _gru(part_ref, h_ref, wih_ref, whh_ref, bih_ref, bhh_ref)
    h_out_ref[...] = hn
    m_out_ref[...] = jnp.dot(hn, wnext_ref[...],
                             preferred_element_type=jnp.float32)


_gru_mm = pl.pallas_call(
    _gru_mm_body,
    grid=(GRID,),
    in_specs=[
        pl.BlockSpec((NC, ROW_BLK, D), lambda i: (0, i, 0)),
        pl.BlockSpec((ROW_BLK, D), lambda i: (i, 0)),
        pl.BlockSpec((D, 3 * D), lambda i: (0, 0)),
        pl.BlockSpec((D, 3 * D), lambda i: (0, 0)),
        pl.BlockSpec((1, 3 * D), lambda i: (0, 0)),
        pl.BlockSpec((1, 3 * D), lambda i: (0, 0)),
        pl.BlockSpec((D, D), lambda i: (0, 0)),
    ],
    out_specs=[
        pl.BlockSpec((ROW_BLK, D), lambda i: (i, 0)),
        pl.BlockSpec((ROW_BLK, D), lambda i: (i, 0)),
    ],
    out_shape=[
        jax.ShapeDtypeStruct((N_NODES, D), jnp.float32),
        jax.ShapeDtypeStruct((N_NODES, D), jnp.float32),
    ],
)


def _gru_pool_body(part_ref, h_ref, wih_ref, whh_ref, bih_ref, bhh_ref,
                   bb_ref, o_ref):
    hn = _gru(part_ref, h_ref, wih_ref, whh_ref, bih_ref, bhh_ref)
    iota = lax.broadcasted_iota(jnp.int32, (ROW_BLK, D), 1)
    onehot = (bb_ref[...] == iota.astype(jnp.float32))
    contrib = lax.dot_general(onehot.astype(jnp.float32), hn,
                              (((0,), (0,)), ((), ())),
                              preferred_element_type=jnp.float32)

    @pl.when(pl.program_id(0) == 0)
    def _():
        o_ref[...] = jnp.zeros_like(o_ref)

    o_ref[...] += contrib


_gru_pool = pl.pallas_call(
    _gru_pool_body,
    grid=(GRID,),
    in_specs=[
        pl.BlockSpec((NC, ROW_BLK, D), lambda i: (0, i, 0)),
        pl.BlockSpec((ROW_BLK, D), lambda i: (i, 0)),
        pl.BlockSpec((D, 3 * D), lambda i: (0, 0)),
        pl.BlockSpec((D, 3 * D), lambda i: (0, 0)),
        pl.BlockSpec((1, 3 * D), lambda i: (0, 0)),
        pl.BlockSpec((1, 3 * D), lambda i: (0, 0)),
        pl.BlockSpec((ROW_BLK, D), lambda i: (i, 0)),
    ],
    out_specs=pl.BlockSpec((D, D), lambda i: (0, 0)),
    out_shape=jax.ShapeDtypeStruct((D, D), jnp.float32),
)


def kernel(x, edge_index, batch, weight, w_ih, w_hh, b_ih, b_hh):
    src = edge_index[0].astype(jnp.int32)
    dst = edge_index[1].astype(jnp.int32)
    wih_t = w_ih.T
    whh_t = w_hh.T
    bih = b_ih.reshape(1, 3 * D)
    bhh = b_hh.reshape(1, 3 * D)
    zeros = jnp.zeros((N_PAD, D), jnp.float32)
    batch_b = jnp.broadcast_to(
        batch.astype(jnp.float32)[:, None], (N_NODES, D))

    sc_aggregate = _make_sc_aggregate()
    h = x
    m = _mm(x, weight[0])
    for i in range(NUM_LAYERS - 1):
        part = sc_aggregate(m, src, dst, zeros)
        h, m = _gru_mm(part, h, wih_t, whh_t, bih, bhh, weight[i + 1])
    part = sc_aggregate(m, src, dst, zeros)
    out_pad = _gru_pool(part, h, wih_t, whh_t, bih, bhh, batch_b)
    return out_pad[:N_GRAPHS]


# speedup vs baseline: 10.9294x; 2.2013x over previous
import functools

import jax
import jax.numpy as jnp
from jax import lax
from jax.experimental import pallas as pl
from jax.experimental.pallas import tpu as pltpu
from jax.experimental.pallas import tpu_sc as plsc

N_NODES = 10000
D = 128
N_EDGES = 320000
N_GRAPHS = 64
NUM_LAYERS = 3

NC = 2
NS = 16
NW = NC * NS
E_PER_W = N_EDGES // NW
CHUNK = 80
STEPS = E_PER_W // CHUNK
N_PAD = 10240
ROWS_PER_TILE = N_PAD // NS

ROW_BLK = 1000
GRID = N_NODES // ROW_BLK


@functools.cache
def _make_sc_aggregate():
    mesh = plsc.VectorSubcoreMesh(core_axis_name="c", subcore_axis_name="s",
                                  num_cores=NC)

    @functools.partial(
        pl.kernel,
        out_type=jax.ShapeDtypeStruct((NC, N_PAD, D), jnp.float32),
        mesh=mesh,
        scratch_types=[
            pltpu.VMEM((3, CHUNK), jnp.int32),
            pltpu.VMEM((3, CHUNK), jnp.int32),
            pltpu.VMEM((2, CHUNK, D), jnp.float32),
            pltpu.VMEM_SHARED((N_PAD, D), jnp.float32),
            pltpu.SemaphoreType.DMA((3,)),
            pltpu.SemaphoreType.DMA((2,)),
            pltpu.SemaphoreType.DMA((2,)),
        ],
    )
    def sc_aggregate(m_hbm, src_hbm, dst_hbm, zeros_hbm, out_hbm,
                     src_v, dst_v, rows_v, acc_sh, isem, gsem, ssem):
        c = lax.axis_index("c")
        s = lax.axis_index("s")
        wid = s * NC + c
        r0 = s * ROWS_PER_TILE
        ebase = wid * E_PER_W

        def idx_desc(k, b):
            off = ebase + k * CHUNK
            return (
                pltpu.make_async_copy(src_hbm.at[pl.ds(off, CHUNK)],
                                      src_v.at[b], isem.at[b]),
                pltpu.make_async_copy(dst_hbm.at[pl.ds(off, CHUNK)],
                                      dst_v.at[b], isem.at[b]),
            )

        def gather_desc(kb3, b):
            return pltpu.make_async_copy(m_hbm.at[src_v.at[kb3]],
                                         rows_v.at[b], gsem.at[b])

        def scatter_desc(kb3, b):
            return pltpu.make_async_copy(rows_v.at[b],
                                         acc_sh.at[dst_v.at[kb3]],
                                         ssem.at[b])

        pltpu.sync_copy(zeros_hbm.at[pl.ds(0, CHUNK)], rows_v.at[0])
        for k in range(ROWS_PER_TILE // CHUNK):
            pltpu.sync_copy(rows_v.at[0],
                            acc_sh.at[pl.ds(r0 + k * CHUNK, CHUNK)])
        plsc.subcore_barrier()

        for d in idx_desc(0, 0):
            d.start()
        for d in idx_desc(1, 1):
            d.start()
        for d in idx_desc(0, 0):
            d.wait()
        gather_desc(0, 0).start()

        def body(k, carry):
            b2 = lax.rem(k, 2)
            nb2 = 1 - b2
            b3 = lax.rem(k, 3)

            @pl.when(k >= 1)
            def _():
                scatter_desc(lax.rem(k - 1, 3), nb2).wait()

            @pl.when(k + 1 < STEPS)
            def _():
                nb3 = lax.rem(k + 1, 3)
                for d in idx_desc(k + 1, nb3):
                    d.wait()
                gather_desc(nb3, nb2).start()

            gather_desc(b3, b2).wait()
            scatter_desc(b3, b2).start(add=True)

            @pl.when(k + 2 < STEPS)
            def _():
                for d in idx_desc(k + 2, lax.rem(k + 2, 3)):
                    d.start()

            return carry

        lax.fori_loop(0, STEPS, body, 0)
        scatter_desc(lax.rem(STEPS - 1, 3), (STEPS - 1) % 2).wait()
        plsc.subcore_barrier()

        for k in range(ROWS_PER_TILE // CHUNK):
            pltpu.sync_copy(acc_sh.at[pl.ds(r0 + k * CHUNK, CHUNK)],
                            rows_v.at[0])
            pltpu.sync_copy(rows_v.at[0],
                            out_hbm.at[c, pl.ds(r0 + k * CHUNK, CHUNK)])

    return sc_aggregate


def _mm_body(x_ref, w_ref, o_ref):
    o_ref[...] = jnp.dot(x_ref[...], w_ref[...],
                         preferred_element_type=jnp.float32)


_mm = pl.pallas_call(
    _mm_body,
    grid=(GRID,),
    in_specs=[
        pl.BlockSpec((ROW_BLK, D), lambda i: (i, 0)),
        pl.BlockSpec((D, D), lambda i: (0, 0)),
    ],
    out_specs=pl.BlockSpec((ROW_BLK, D), lambda i: (i, 0)),
    out_shape=jax.ShapeDtypeStruct((N_NODES, D), jnp.float32),
)


def _gru(part_ref, h_ref, wih_ref, whh_ref, bih_ref, bhh_ref):
    agg = part_ref[0] + part_ref[1]
    h = h_ref[...]
    gi = jnp.dot(agg, wih_ref[...], preferred_element_type=jnp.float32)
    gi = gi + bih_ref[...]
    gh = jnp.dot(h, whh_ref[...], preferred_element_type=jnp.float32)
    gh = gh + bhh_ref[...]
    r = jax.nn.sigmoid(gi[:, :D] + gh[:, :D])
    z = jax.nn.sigmoid(gi[:, D:2 * D] + gh[:, D:2 * D])
    n = jnp.tanh(gi[:, 2 * D:] + r * gh[:, 2 * D:])
    return (1.0 - z) * n + z * h


def _gru_mm_body(part_ref, h_ref, wih_ref, whh_ref, bih_ref, bhh_ref,
                 wnext_ref, h_out_ref, m_out_ref):
    hn = _gru(part_ref, h_ref, wih_ref, whh_ref, bih_ref, bhh_ref)
    h_out_ref[...] = hn
    m_out_ref[...] = jnp.dot(hn, wnext_ref[...],
                             preferred_element_type=jnp.float32)


_gru_mm = pl.pallas_call(
    _gru_mm_body,
    grid=(GRID,),
    in_specs=[
        pl.BlockSpec((NC, ROW_BLK, D), lambda i: (0, i, 0)),
        pl.BlockSpec((ROW_BLK, D), lambda i: (i, 0)),
        pl.BlockSpec((D, 3 * D), lambda i: (0, 0)),
        pl.BlockSpec((D, 3 * D), lambda i: (0, 0)),
        pl.BlockSpec((1, 3 * D), lambda i: (0, 0)),
        pl.BlockSpec((1, 3 * D), lambda i: (0, 0)),
        pl.BlockSpec((D, D), lambda i: (0, 0)),
    ],
    out_specs=[
        pl.BlockSpec((ROW_BLK, D), lambda i: (i, 0)),
        pl.BlockSpec((ROW_BLK, D), lambda i: (i, 0)),
    ],
    out_shape=[
        jax.ShapeDtypeStruct((N_NODES, D), jnp.float32),
        jax.ShapeDtypeStruct((N_NODES, D), jnp.float32),
    ],
)


def _gru_pool_body(part_ref, h_ref, wih_ref, whh_ref, bih_ref, bhh_ref,
                   bb_ref, o_ref):
    hn = _gru(part_ref, h_ref, wih_ref, whh_ref, bih_ref, bhh_ref)
    iota = lax.broadcasted_iota(jnp.int32, (ROW_BLK, D), 1)
    onehot = (bb_ref[...] == iota.astype(jnp.float32))
    contrib = lax.dot_general(onehot.astype(jnp.float32), hn,
                              (((0,), (0,)), ((), ())),
                              preferred_element_type=jnp.float32)

    @pl.when(pl.program_id(0) == 0)
    def _():
        o_ref[...] = jnp.zeros_like(o_ref)

    o_ref[...] += contrib


_gru_pool = pl.pallas_call(
    _gru_pool_body,
    grid=(GRID,),
    in_specs=[
        pl.BlockSpec((NC, ROW_BLK, D), lambda i: (0, i, 0)),
        pl.BlockSpec((ROW_BLK, D), lambda i: (i, 0)),
        pl.BlockSpec((D, 3 * D), lambda i: (0, 0)),
        pl.BlockSpec((D, 3 * D), lambda i: (0, 0)),
        pl.BlockSpec((1, 3 * D), lambda i: (0, 0)),
        pl.BlockSpec((1, 3 * D), lambda i: (0, 0)),
        pl.BlockSpec((ROW_BLK, D), lambda i: (i, 0)),
    ],
    out_specs=pl.BlockSpec((D, D), lambda i: (0, 0)),
    out_shape=jax.ShapeDtypeStruct((D, D), jnp.float32),
)


def kernel(x, edge_index, batch, weight, w_ih, w_hh, b_ih, b_hh):
    src = edge_index[0].astype(jnp.int32)
    dst = edge_index[1].astype(jnp.int32)
    wih_t = w_ih.T
    whh_t = w_hh.T
    bih = b_ih.reshape(1, 3 * D)
    bhh = b_hh.reshape(1, 3 * D)
    zeros = jnp.zeros((N_PAD, D), jnp.float32)
    batch_b = jnp.broadcast_to(
        batch.astype(jnp.float32)[:, None], (N_NODES, D))

    sc_aggregate = _make_sc_aggregate()
    h = x
    m = _mm(x, weight[0])
    for i in range(NUM_LAYERS - 1):
        part = sc_aggregate(m, src, dst, zeros)
        h, m = _gru_mm(part, h, wih_t, whh_t, bih, bhh, weight[i + 1])
    part = sc_aggregate(m, src, dst, zeros)
    out_pad = _gru_pool(part, h, wih_t, whh_t, bih, bhh, batch_b)
    return out_pad[:N_GRAPHS]


# speedup vs baseline: 12.1289x; 1.1098x over previous
import functools

import jax
import jax.numpy as jnp
from jax import lax
from jax.experimental import pallas as pl
from jax.experimental.pallas import tpu as pltpu
from jax.experimental.pallas import tpu_sc as plsc

N_NODES = 10000
D = 128
N_EDGES = 320000
N_GRAPHS = 64
NUM_LAYERS = 3

NC = 2
NS = 16
NW = NC * NS
E_PER_W = N_EDGES // NW
CHUNK = 128
STEPS = E_PER_W // CHUNK
TAIL = E_PER_W - STEPS * CHUNK
N_PAD = 10240
ROWS_PER_TILE = N_PAD // NS

ROW_BLK = 1000
GRID = N_NODES // ROW_BLK


@functools.cache
def _make_sc_aggregate():
    mesh = plsc.VectorSubcoreMesh(core_axis_name="c", subcore_axis_name="s",
                                  num_cores=NC)

    @functools.partial(
        pl.kernel,
        out_type=jax.ShapeDtypeStruct((NC, N_PAD, D), jnp.float32),
        mesh=mesh,
        scratch_types=[
            pltpu.VMEM((3, CHUNK), jnp.int32),
            pltpu.VMEM((3, CHUNK), jnp.int32),
            pltpu.VMEM((2, CHUNK, D), jnp.float32),
            pltpu.VMEM_SHARED((N_PAD, D), jnp.float32),
            pltpu.VMEM((TAIL,), jnp.int32),
            pltpu.VMEM((TAIL,), jnp.int32),
            pltpu.VMEM((TAIL, D), jnp.float32),
            pltpu.SemaphoreType.DMA((3,)),
            pltpu.SemaphoreType.DMA((2,)),
            pltpu.SemaphoreType.DMA((2,)),
            pltpu.SemaphoreType.DMA,
        ],
    )
    def sc_aggregate(m_hbm, src_hbm, dst_hbm, zeros_hbm, out_hbm,
                     src_v, dst_v, rows_v, acc_sh, src_t, dst_t, rows_t,
                     isem, gsem, ssem, tsem):
        c = lax.axis_index("c")
        s = lax.axis_index("s")
        wid = s * NC + c
        r0 = s * ROWS_PER_TILE
        ebase = wid * E_PER_W

        def idx_desc(k, b):
            off = ebase + k * CHUNK
            return (
                pltpu.make_async_copy(src_hbm.at[pl.ds(off, CHUNK)],
                                      src_v.at[b], isem.at[b]),
                pltpu.make_async_copy(dst_hbm.at[pl.ds(off, CHUNK)],
                                      dst_v.at[b], isem.at[b]),
            )

        def gather_desc(kb3, b):
            return pltpu.make_async_copy(m_hbm.at[src_v.at[kb3]],
                                         rows_v.at[b], gsem.at[b])

        def scatter_desc(kb3, b):
            return pltpu.make_async_copy(rows_v.at[b],
                                         acc_sh.at[dst_v.at[kb3]],
                                         ssem.at[b])

        pltpu.sync_copy(zeros_hbm.at[pl.ds(0, CHUNK)], rows_v.at[0])
        for k in range(ROWS_PER_TILE // CHUNK):
            pltpu.sync_copy(rows_v.at[0],
                            acc_sh.at[pl.ds(r0 + k * CHUNK, CHUNK)])
        plsc.subcore_barrier()

        for d in idx_desc(0, 0):
            d.start()
        for d in idx_desc(1, 1):
            d.start()
        for d in idx_desc(0, 0):
            d.wait()
        gather_desc(0, 0).start()

        def body(k, carry):
            b2 = lax.rem(k, 2)
            nb2 = 1 - b2
            b3 = lax.rem(k, 3)

            @pl.when(k >= 1)
            def _():
                scatter_desc(lax.rem(k - 1, 3), nb2).wait()

            @pl.when(k + 1 < STEPS)
            def _():
                nb3 = lax.rem(k + 1, 3)
                for d in idx_desc(k + 1, nb3):
                    d.wait()
                gather_desc(nb3, nb2).start()

            gather_desc(b3, b2).wait()
            scatter_desc(b3, b2).start(add=True)

            @pl.when(k + 2 < STEPS)
            def _():
                for d in idx_desc(k + 2, lax.rem(k + 2, 3)):
                    d.start()

            return carry

        lax.fori_loop(0, STEPS, body, 0)
        toff = ebase + STEPS * CHUNK
        pltpu.sync_copy(src_hbm.at[pl.ds(toff, TAIL)], src_t)
        pltpu.sync_copy(dst_hbm.at[pl.ds(toff, TAIL)], dst_t)
        pltpu.async_copy(m_hbm.at[src_t], rows_t, tsem).wait()
        scatter_desc(lax.rem(STEPS - 1, 3), (STEPS - 1) % 2).wait()
        pltpu.async_copy(rows_t, acc_sh.at[dst_t], tsem, add=True).wait()
        plsc.subcore_barrier()

        for k in range(ROWS_PER_TILE // CHUNK):
            pltpu.sync_copy(acc_sh.at[pl.ds(r0 + k * CHUNK, CHUNK)],
                            rows_v.at[0])
            pltpu.sync_copy(rows_v.at[0],
                            out_hbm.at[c, pl.ds(r0 + k * CHUNK, CHUNK)])

    return sc_aggregate


def _mm_body(x_ref, w_ref, o_ref):
    o_ref[...] = jnp.dot(x_ref[...], w_ref[...],
                         preferred_element_type=jnp.float32)


_mm = pl.pallas_call(
    _mm_body,
    grid=(GRID,),
    in_specs=[
        pl.BlockSpec((ROW_BLK, D), lambda i: (i, 0)),
        pl.BlockSpec((D, D), lambda i: (0, 0)),
    ],
    out_specs=pl.BlockSpec((ROW_BLK, D), lambda i: (i, 0)),
    out_shape=jax.ShapeDtypeStruct((N_NODES, D), jnp.float32),
)


def _gru(part_ref, h_ref, wih_ref, whh_ref, bih_ref, bhh_ref):
    agg = part_ref[0] + part_ref[1]
    h = h_ref[...]
    gi = jnp.dot(agg, wih_ref[...], preferred_element_type=jnp.float32)
    gi = gi + bih_ref[...]
    gh = jnp.dot(h, whh_ref[...], preferred_element_type=jnp.float32)
    gh = gh + bhh_ref[...]
    r = jax.nn.sigmoid(gi[:, :D] + gh[:, :D])
    z = jax.nn.sigmoid(gi[:, D:2 * D] + gh[:, D:2 * D])
    n = jnp.tanh(gi[:, 2 * D:] + r * gh[:, 2 * D:])
    return (1.0 - z) * n + z * h


def _gru_mm_body(part_ref, h_ref, wih_ref, whh_ref, bih_ref, bhh_ref,
                 wnext_ref, h_out_ref, m_out_ref):
    hn = _gru(part_ref, h_ref, wih_ref, whh_ref, bih_ref, bhh_ref)
    h_out_ref[...] = hn
    m_out_ref[...] = jnp.dot(hn, wnext_ref[...],
                             preferred_element_type=jnp.float32)


_gru_mm = pl.pallas_call(
    _gru_mm_body,
    grid=(GRID,),
    in_specs=[
        pl.BlockSpec((NC, ROW_BLK, D), lambda i: (0, i, 0)),
        pl.BlockSpec((ROW_BLK, D), lambda i: (i, 0)),
        pl.BlockSpec((D, 3 * D), lambda i: (0, 0)),
        pl.BlockSpec((D, 3 * D), lambda i: (0, 0)),
        pl.BlockSpec((1, 3 * D), lambda i: (0, 0)),
        pl.BlockSpec((1, 3 * D), lambda i: (0, 0)),
        pl.BlockSpec((D, D), lambda i: (0, 0)),
    ],
    out_specs=[
        pl.BlockSpec((ROW_BLK, D), lambda i: (i, 0)),
        pl.BlockSpec((ROW_BLK, D), lambda i: (i, 0)),
    ],
    out_shape=[
        jax.ShapeDtypeStruct((N_NODES, D), jnp.float32),
        jax.ShapeDtypeStruct((N_NODES, D), jnp.float32),
    ],
)


def _gru_pool_body(part_ref, h_ref, wih_ref, whh_ref, bih_ref, bhh_ref,
                   bb_ref, o_ref):
    hn = _gru(part_ref, h_ref, wih_ref, whh_ref, bih_ref, bhh_ref)
    iota = lax.broadcasted_iota(jnp.int32, (ROW_BLK, D), 1)
    onehot = (bb_ref[...] == iota.astype(jnp.float32))
    contrib = lax.dot_general(onehot.astype(jnp.float32), hn,
                              (((0,), (0,)), ((), ())),
                              preferred_element_type=jnp.float32)

    @pl.when(pl.program_id(0) == 0)
    def _():
        o_ref[...] = jnp.zeros_like(o_ref)

    o_ref[...] += contrib


_gru_pool = pl.pallas_call(
    _gru_pool_body,
    grid=(GRID,),
    in_specs=[
        pl.BlockSpec((NC, ROW_BLK, D), lambda i: (0, i, 0)),
        pl.BlockSpec((ROW_BLK, D), lambda i: (i, 0)),
        pl.BlockSpec((D, 3 * D), lambda i: (0, 0)),
        pl.BlockSpec((D, 3 * D), lambda i: (0, 0)),
        pl.BlockSpec((1, 3 * D), lambda i: (0, 0)),
        pl.BlockSpec((1, 3 * D), lambda i: (0, 0)),
        pl.BlockSpec((ROW_BLK, D), lambda i: (i, 0)),
    ],
    out_specs=pl.BlockSpec((D, D), lambda i: (0, 0)),
    out_shape=jax.ShapeDtypeStruct((D, D), jnp.float32),
)


def kernel(x, edge_index, batch, weight, w_ih, w_hh, b_ih, b_hh):
    src = edge_index[0].astype(jnp.int32)
    dst = edge_index[1].astype(jnp.int32)
    wih_t = w_ih.T
    whh_t = w_hh.T
    bih = b_ih.reshape(1, 3 * D)
    bhh = b_hh.reshape(1, 3 * D)
    zeros = jnp.zeros((N_PAD, D), jnp.float32)
    batch_b = jnp.broadcast_to(
        batch.astype(jnp.float32)[:, None], (N_NODES, D))

    sc_aggregate = _make_sc_aggregate()
    h = x
    m = _mm(x, weight[0])
    for i in range(NUM_LAYERS - 1):
        part = sc_aggregate(m, src, dst, zeros)
        h, m = _gru_mm(part, h, wih_t, whh_t, bih, bhh, weight[i + 1])
    part = sc_aggregate(m, src, dst, zeros)
    out_pad = _gru_pool(part, h, wih_t, whh_t, bih, bhh, batch_b)
    return out_pad[:N_GRAPHS]


# speedup vs baseline: 12.1589x; 1.0025x over previous
import functools

import jax
import jax.numpy as jnp
from jax import lax
from jax.experimental import pallas as pl
from jax.experimental.pallas import tpu as pltpu
from jax.experimental.pallas import tpu_sc as plsc

N_NODES = 10000
D = 128
N_EDGES = 320000
N_GRAPHS = 64
NUM_LAYERS = 3

NC = 2
NS = 16
NW = NC * NS
CHUNK = 128
STEPS = 78
SUPER = 6
NSUPER = STEPS // SUPER
E_MAIN = STEPS * CHUNK
N_EXTRA = (N_EDGES - NW * E_MAIN) // CHUNK
N_PAD = 10240
ROWS_PER_TILE = N_PAD // NS

ROW_BLK = 1000
GRID = N_NODES // ROW_BLK


@functools.cache
def _make_sc_aggregate():
    mesh = plsc.VectorSubcoreMesh(core_axis_name="c", subcore_axis_name="s",
                                  num_cores=NC)

    @functools.partial(
        pl.kernel,
        out_type=jax.ShapeDtypeStruct((NC, N_PAD, D), jnp.float32),
        mesh=mesh,
        scratch_types=[
            pltpu.VMEM((4, SUPER * CHUNK), jnp.int32),
            pltpu.VMEM((4, SUPER * CHUNK), jnp.int32),
            pltpu.VMEM((2, CHUNK, D), jnp.float32),
            pltpu.VMEM_SHARED((N_PAD, D), jnp.float32),
            pltpu.VMEM((CHUNK,), jnp.int32),
            pltpu.VMEM((CHUNK,), jnp.int32),
            pltpu.SemaphoreType.DMA((4,)),
            pltpu.SemaphoreType.DMA((2,)),
            pltpu.SemaphoreType.DMA((2,)),
            pltpu.SemaphoreType.DMA,
        ],
    )
    def sc_aggregate(m_hbm, src_hbm, dst_hbm, zeros_hbm, out_hbm,
                     src_v, dst_v, rows_v, acc_sh, src_t, dst_t,
                     isem, gsem, ssem, tsem):
        c = lax.axis_index("c")
        s = lax.axis_index("s")
        wid = s * NC + c
        r0 = s * ROWS_PER_TILE
        ebase = wid * E_MAIN

        def idx_desc(g, b4):
            off = ebase + g * SUPER * CHUNK
            return (
                pltpu.make_async_copy(
                    src_hbm.at[pl.ds(off, SUPER * CHUNK)],
                    src_v.at[b4], isem.at[b4]),
                pltpu.make_async_copy(
                    dst_hbm.at[pl.ds(off, SUPER * CHUNK)],
                    dst_v.at[b4], isem.at[b4]),
            )

        def gather_desc(b4, j, b2):
            return pltpu.make_async_copy(
                m_hbm.at[src_v.at[b4, pl.ds(j * CHUNK, CHUNK)]],
                rows_v.at[b2], gsem.at[b2])

        def scatter_desc(b4, j, b2):
            return pltpu.make_async_copy(
                rows_v.at[b2],
                acc_sh.at[dst_v.at[b4, pl.ds(j * CHUNK, CHUNK)]],
                ssem.at[b2])

        pltpu.sync_copy(zeros_hbm.at[pl.ds(0, CHUNK)], rows_v.at[0])
        zdescs = [
            pltpu.make_async_copy(
                rows_v.at[0], acc_sh.at[pl.ds(r0 + k * CHUNK, CHUNK)],
                isem.at[0])
            for k in range(ROWS_PER_TILE // CHUNK)
        ]
        for dsc in zdescs:
            dsc.start()
        for dsc in zdescs:
            dsc.wait()
        plsc.subcore_barrier()

        for d in idx_desc(0, 0):
            d.start()
        for d in idx_desc(1, 1):
            d.start()
        for d in idx_desc(0, 0):
            d.wait()
        gather_desc(0, 0, 0).start()

        def body(k, carry):
            b2 = lax.rem(k, 2)
            nb2 = 1 - b2
            g = lax.div(k, SUPER)
            j = lax.rem(k, SUPER)
            b4 = lax.rem(g, 4)

            @pl.when(k >= 1)
            def _():
                g_1 = lax.div(k - 1, SUPER)
                scatter_desc(lax.rem(g_1, 4), lax.rem(k - 1, SUPER),
                             nb2).wait()

            @pl.when(k + 1 < STEPS)
            def _():
                g1 = lax.div(k + 1, SUPER)
                j1 = lax.rem(k + 1, SUPER)
                b41 = lax.rem(g1, 4)

                @pl.when(j1 == 0)
                def _():
                    for d in idx_desc(g1, b41):
                        d.wait()

                gather_desc(b41, j1, nb2).start()

            gather_desc(b4, j, b2).wait()
            scatter_desc(b4, j, b2).start(add=True)

            @pl.when(jnp.logical_and(j == 0, g + 2 < NSUPER))
            def _():
                for d in idx_desc(g + 2, lax.rem(g + 2, 4)):
                    d.start()

            return carry

        lax.fori_loop(0, STEPS, body, 0)
        scatter_desc((STEPS - 1) // SUPER % 4, (STEPS - 1) % SUPER,
                     (STEPS - 1) % 2).wait()

        @pl.when(wid < N_EXTRA)
        def _():
            toff = NW * E_MAIN + wid * CHUNK
            pltpu.sync_copy(src_hbm.at[pl.ds(toff, CHUNK)], src_t)
            pltpu.sync_copy(dst_hbm.at[pl.ds(toff, CHUNK)], dst_t)
            pltpu.async_copy(m_hbm.at[src_t], rows_v.at[0], tsem).wait()
            pltpu.async_copy(rows_v.at[0], acc_sh.at[dst_t], tsem,
                             add=True).wait()

        plsc.subcore_barrier()

        npiece = ROWS_PER_TILE // CHUNK

        def out_desc(k, b):
            return pltpu.make_async_copy(
                rows_v.at[b], out_hbm.at[c, pl.ds(r0 + k * CHUNK, CHUNK)],
                ssem.at[b])

        for k in range(npiece):
            b = k % 2
            if k >= 2:
                out_desc(k - 2, b).wait()
            pltpu.sync_copy(acc_sh.at[pl.ds(r0 + k * CHUNK, CHUNK)],
                            rows_v.at[b])
            out_desc(k, b).start()
        for k in (npiece - 2, npiece - 1):
            out_desc(k, k % 2).wait()

    return sc_aggregate


def _mm_body(x_ref, w_ref, o_ref):
    o_ref[...] = jnp.dot(x_ref[...], w_ref[...],
                         preferred_element_type=jnp.float32)


_mm = pl.pallas_call(
    _mm_body,
    grid=(GRID,),
    in_specs=[
        pl.BlockSpec((ROW_BLK, D), lambda i: (i, 0)),
        pl.BlockSpec((D, D), lambda i: (0, 0)),
    ],
    out_specs=pl.BlockSpec((ROW_BLK, D), lambda i: (i, 0)),
    out_shape=jax.ShapeDtypeStruct((N_NODES, D), jnp.float32),
)


def _gru(part_ref, h_ref, wih_ref, whh_ref, bih_ref, bhh_ref):
    agg = part_ref[0] + part_ref[1]
    h = h_ref[...]
    gi = jnp.dot(agg, wih_ref[...], preferred_element_type=jnp.float32)
    gi = gi + bih_ref[...]
    gh = jnp.dot(h, whh_ref[...], preferred_element_type=jnp.float32)
    gh = gh + bhh_ref[...]
    r = jax.nn.sigmoid(gi[:, :D] + gh[:, :D])
    z = jax.nn.sigmoid(gi[:, D:2 * D] + gh[:, D:2 * D])
    n = jnp.tanh(gi[:, 2 * D:] + r * gh[:, 2 * D:])
    return (1.0 - z) * n + z * h


def _gru_mm_body(part_ref, h_ref, wih_ref, whh_ref, bih_ref, bhh_ref,
                 wnext_ref, h_out_ref, m_out_ref):
    hn = _gru(part_ref, h_ref, wih_ref, whh_ref, bih_ref, bhh_ref)
    h_out_ref[...] = hn
    m_out_ref[...] = jnp.dot(hn, wnext_ref[...],
                             preferred_element_type=jnp.float32)


_gru_mm = pl.pallas_call(
    _gru_mm_body,
    grid=(GRID,),
    in_specs=[
        pl.BlockSpec((NC, ROW_BLK, D), lambda i: (0, i, 0)),
        pl.BlockSpec((ROW_BLK, D), lambda i: (i, 0)),
        pl.BlockSpec((D, 3 * D), lambda i: (0, 0)),
        pl.BlockSpec((D, 3 * D), lambda i: (0, 0)),
        pl.BlockSpec((1, 3 * D), lambda i: (0, 0)),
        pl.BlockSpec((1, 3 * D), lambda i: (0, 0)),
        pl.BlockSpec((D, D), lambda i: (0, 0)),
    ],
    out_specs=[
        pl.BlockSpec((ROW_BLK, D), lambda i: (i, 0)),
        pl.BlockSpec((ROW_BLK, D), lambda i: (i, 0)),
    ],
    out_shape=[
        jax.ShapeDtypeStruct((N_NODES, D), jnp.float32),
        jax.ShapeDtypeStruct((N_NODES, D), jnp.float32),
    ],
)


def _gru_pool_body(part_ref, h_ref, wih_ref, whh_ref, bih_ref, bhh_ref,
                   bb_ref, o_ref):
    hn = _gru(part_ref, h_ref, wih_ref, whh_ref, bih_ref, bhh_ref)
    iota = lax.broadcasted_iota(jnp.int32, (ROW_BLK, D), 1)
    onehot = (bb_ref[...] == iota.astype(jnp.float32))
    contrib = lax.dot_general(onehot.astype(jnp.float32), hn,
                              (((0,), (0,)), ((), ())),
                              preferred_element_type=jnp.float32)

    @pl.when(pl.program_id(0) == 0)
    def _():
        o_ref[...] = jnp.zeros_like(o_ref)

    o_ref[...] += contrib


_gru_pool = pl.pallas_call(
    _gru_pool_body,
    grid=(GRID,),
    in_specs=[
        pl.BlockSpec((NC, ROW_BLK, D), lambda i: (0, i, 0)),
        pl.BlockSpec((ROW_BLK, D), lambda i: (i, 0)),
        pl.BlockSpec((D, 3 * D), lambda i: (0, 0)),
        pl.BlockSpec((D, 3 * D), lambda i: (0, 0)),
        pl.BlockSpec((1, 3 * D), lambda i: (0, 0)),
        pl.BlockSpec((1, 3 * D), lambda i: (0, 0)),
        pl.BlockSpec((ROW_BLK, D), lambda i: (i, 0)),
    ],
    out_specs=pl.BlockSpec((D, D), lambda i: (0, 0)),
    out_shape=jax.ShapeDtypeStruct((D, D), jnp.float32),
)


def kernel(x, edge_index, batch, weight, w_ih, w_hh, b_ih, b_hh):
    src = edge_index[0].astype(jnp.int32)
    dst = edge_index[1].astype(jnp.int32)
    wih_t = w_ih.T
    whh_t = w_hh.T
    bih = b_ih.reshape(1, 3 * D)
    bhh = b_hh.reshape(1, 3 * D)
    zeros = jnp.zeros((N_PAD, D), jnp.float32)
    batch_b = jnp.broadcast_to(
        batch.astype(jnp.float32)[:, None], (N_NODES, D))

    sc_aggregate = _make_sc_aggregate()
    h = x
    m = _mm(x, weight[0])
    for i in range(NUM_LAYERS - 1):
        part = sc_aggregate(m, src, dst, zeros)
        h, m = _gru_mm(part, h, wih_t, whh_t, bih, bhh, weight[i + 1])
    part = sc_aggregate(m, src, dst, zeros)
    out_pad = _gru_pool(part, h, wih_t, whh_t, bih, bhh, batch_b)
    return out_pad[:N_GRAPHS]


# speedup vs baseline: 12.2064x; 1.0039x over previous
import functools

import jax
import jax.numpy as jnp
from jax import lax
from jax.experimental import pallas as pl
from jax.experimental.pallas import tpu as pltpu
from jax.experimental.pallas import tpu_sc as plsc

N_NODES = 10000
D = 128
N_EDGES = 320000
N_GRAPHS = 64
NUM_LAYERS = 3

NC = 2
NS = 16
NW = NC * NS
CHUNK = 128
STEPS = 78
SUPER = 6
NSUPER = STEPS // SUPER
E_MAIN = STEPS * CHUNK
N_EXTRA = (N_EDGES - NW * E_MAIN) // CHUNK
N_PAD = 10240
ROWS_PER_TILE = N_PAD // NS

ROW_BLK = 1000
GRID = N_NODES // ROW_BLK


@functools.cache
def _make_sc_aggregate():
    mesh = plsc.VectorSubcoreMesh(core_axis_name="c", subcore_axis_name="s",
                                  num_cores=NC)

    @functools.partial(
        pl.kernel,
        out_type=jax.ShapeDtypeStruct((NC, N_PAD, D), jnp.float32),
        mesh=mesh,
        scratch_types=[
            pltpu.VMEM((4, SUPER * CHUNK), jnp.int32),
            pltpu.VMEM((4, SUPER * CHUNK), jnp.int32),
            pltpu.VMEM((2, CHUNK, D), jnp.float32),
            pltpu.VMEM_SHARED((N_PAD, D), jnp.float32),
            pltpu.VMEM((CHUNK,), jnp.int32),
            pltpu.VMEM((CHUNK,), jnp.int32),
            pltpu.SemaphoreType.DMA((4,)),
            pltpu.SemaphoreType.DMA((2,)),
            pltpu.SemaphoreType.DMA((2,)),
            pltpu.SemaphoreType.DMA,
        ],
    )
    def sc_aggregate(m_hbm, src_hbm, dst_hbm, zeros_hbm, out_hbm,
                     src_v, dst_v, rows_v, acc_sh, src_t, dst_t,
                     isem, gsem, ssem, tsem):
        c = lax.axis_index("c")
        s = lax.axis_index("s")
        wid = s * NC + c
        r0 = s * ROWS_PER_TILE
        ebase = wid * E_MAIN

        def idx_desc(g, b4):
            off = ebase + g * SUPER * CHUNK
            return (
                pltpu.make_async_copy(
                    src_hbm.at[pl.ds(off, SUPER * CHUNK)],
                    src_v.at[b4], isem.at[b4]),
                pltpu.make_async_copy(
                    dst_hbm.at[pl.ds(off, SUPER * CHUNK)],
                    dst_v.at[b4], isem.at[b4]),
            )

        def gather_desc(b4, j, b2):
            return pltpu.make_async_copy(
                m_hbm.at[src_v.at[b4, pl.ds(j * CHUNK, CHUNK)]],
                rows_v.at[b2], gsem.at[b2])

        def scatter_desc(b4, j, b2):
            return pltpu.make_async_copy(
                rows_v.at[b2],
                acc_sh.at[dst_v.at[b4, pl.ds(j * CHUNK, CHUNK)]],
                ssem.at[b2])

        pltpu.sync_copy(zeros_hbm.at[pl.ds(0, CHUNK)], rows_v.at[0])
        zdescs = [
            pltpu.make_async_copy(
                rows_v.at[0], acc_sh.at[pl.ds(r0 + k * CHUNK, CHUNK)],
                isem.at[0])
            for k in range(ROWS_PER_TILE // CHUNK)
        ]
        for dsc in zdescs:
            dsc.start()
        for dsc in zdescs:
            dsc.wait()
        plsc.subcore_barrier()

        for d in idx_desc(0, 0):
            d.start()
        for d in idx_desc(1, 1):
            d.start()
        for d in idx_desc(0, 0):
            d.wait()
        gather_desc(0, 0, 0).start()

        def body(k, carry):
            b2 = lax.rem(k, 2)
            nb2 = 1 - b2
            g = lax.div(k, SUPER)
            j = lax.rem(k, SUPER)
            b4 = lax.rem(g, 4)

            @pl.when(k >= 1)
            def _():
                g_1 = lax.div(k - 1, SUPER)
                scatter_desc(lax.rem(g_1, 4), lax.rem(k - 1, SUPER),
                             nb2).wait()

            @pl.when(k + 1 < STEPS)
            def _():
                g1 = lax.div(k + 1, SUPER)
                j1 = lax.rem(k + 1, SUPER)
                b41 = lax.rem(g1, 4)

                @pl.when(j1 == 0)
                def _():
                    for d in idx_desc(g1, b41):
                        d.wait()

                gather_desc(b41, j1, nb2).start()

            gather_desc(b4, j, b2).wait()
            scatter_desc(b4, j, b2).start(add=True)

            @pl.when(jnp.logical_and(j == 0, g + 2 < NSUPER))
            def _():
                for d in idx_desc(g + 2, lax.rem(g + 2, 4)):
                    d.start()

            return carry

        lax.fori_loop(0, STEPS, body, 0)
        scatter_desc((STEPS - 1) // SUPER % 4, (STEPS - 1) % SUPER,
                     (STEPS - 1) % 2).wait()

        @pl.when(wid < N_EXTRA)
        def _():
            toff = NW * E_MAIN + wid * CHUNK
            pltpu.sync_copy(src_hbm.at[pl.ds(toff, CHUNK)], src_t)
            pltpu.sync_copy(dst_hbm.at[pl.ds(toff, CHUNK)], dst_t)
            pltpu.async_copy(m_hbm.at[src_t], rows_v.at[0], tsem).wait()
            pltpu.async_copy(rows_v.at[0], acc_sh.at[dst_t], tsem,
                             add=True).wait()

        plsc.subcore_barrier()

        npiece = ROWS_PER_TILE // CHUNK

        def out_desc(k, b):
            return pltpu.make_async_copy(
                rows_v.at[b], out_hbm.at[c, pl.ds(r0 + k * CHUNK, CHUNK)],
                ssem.at[b])

        for k in range(npiece):
            b = k % 2
            if k >= 2:
                out_desc(k - 2, b).wait()
            pltpu.sync_copy(acc_sh.at[pl.ds(r0 + k * CHUNK, CHUNK)],
                            rows_v.at[b])
            out_desc(k, b).start()
        for k in (npiece - 2, npiece - 1):
            out_desc(k, k % 2).wait()

    return sc_aggregate


def _bf_dot(a, b):
    return jnp.dot(a.astype(jnp.bfloat16), b.astype(jnp.bfloat16),
                   preferred_element_type=jnp.float32)


def _mm_body(x_ref, w_ref, o_ref):
    o_ref[...] = _bf_dot(x_ref[...], w_ref[...])


_mm = pl.pallas_call(
    _mm_body,
    grid=(GRID,),
    in_specs=[
        pl.BlockSpec((ROW_BLK, D), lambda i: (i, 0)),
        pl.BlockSpec((D, D), lambda i: (0, 0)),
    ],
    out_specs=pl.BlockSpec((ROW_BLK, D), lambda i: (i, 0)),
    out_shape=jax.ShapeDtypeStruct((N_NODES, D), jnp.float32),
)


def _gru(part_ref, h_ref, wih_ref, whh_ref, bih_ref, bhh_ref):
    agg = part_ref[0] + part_ref[1]
    h = h_ref[...]
    gi = _bf_dot(agg, wih_ref[...]) + bih_ref[...]
    gh = _bf_dot(h, whh_ref[...]) + bhh_ref[...]
    r = jax.nn.sigmoid(gi[:, :D] + gh[:, :D])
    z = jax.nn.sigmoid(gi[:, D:2 * D] + gh[:, D:2 * D])
    n = jnp.tanh(gi[:, 2 * D:] + r * gh[:, 2 * D:])
    return (1.0 - z) * n + z * h


def _gru_mm_body(part_ref, h_ref, wih_ref, whh_ref, bih_ref, bhh_ref,
                 wnext_ref, h_out_ref, m_out_ref):
    hn = _gru(part_ref, h_ref, wih_ref, whh_ref, bih_ref, bhh_ref)
    h_out_ref[...] = hn
    m_out_ref[...] = _bf_dot(hn, wnext_ref[...])


_gru_mm = pl.pallas_call(
    _gru_mm_body,
    grid=(GRID,),
    in_specs=[
        pl.BlockSpec((NC, ROW_BLK, D), lambda i: (0, i, 0)),
        pl.BlockSpec((ROW_BLK, D), lambda i: (i, 0)),
        pl.BlockSpec((D, 3 * D), lambda i: (0, 0)),
        pl.BlockSpec((D, 3 * D), lambda i: (0, 0)),
        pl.BlockSpec((1, 3 * D), lambda i: (0, 0)),
        pl.BlockSpec((1, 3 * D), lambda i: (0, 0)),
        pl.BlockSpec((D, D), lambda i: (0, 0)),
    ],
    out_specs=[
        pl.BlockSpec((ROW_BLK, D), lambda i: (i, 0)),
        pl.BlockSpec((ROW_BLK, D), lambda i: (i, 0)),
    ],
    out_shape=[
        jax.ShapeDtypeStruct((N_NODES, D), jnp.float32),
        jax.ShapeDtypeStruct((N_NODES, D), jnp.float32),
    ],
)


def _gru_pool_body(part_ref, h_ref, wih_ref, whh_ref, bih_ref, bhh_ref,
                   bb_ref, o_ref):
    hn = _gru(part_ref, h_ref, wih_ref, whh_ref, bih_ref, bhh_ref)
    iota = lax.broadcasted_iota(jnp.int32, (ROW_BLK, D), 1)
    onehot = (bb_ref[...] == iota.astype(jnp.float32))
    contrib = lax.dot_general(onehot.astype(jnp.float32), hn,
                              (((0,), (0,)), ((), ())),
                              preferred_element_type=jnp.float32)

    @pl.when(pl.program_id(0) == 0)
    def _():
        o_ref[...] = jnp.zeros_like(o_ref)

    o_ref[...] += contrib


_gru_pool = pl.pallas_call(
    _gru_pool_body,
    grid=(GRID,),
    in_specs=[
        pl.BlockSpec((NC, ROW_BLK, D), lambda i: (0, i, 0)),
        pl.BlockSpec((ROW_BLK, D), lambda i: (i, 0)),
        pl.BlockSpec((D, 3 * D), lambda i: (0, 0)),
        pl.BlockSpec((D, 3 * D), lambda i: (0, 0)),
        pl.BlockSpec((1, 3 * D), lambda i: (0, 0)),
        pl.BlockSpec((1, 3 * D), lambda i: (0, 0)),
        pl.BlockSpec((ROW_BLK, D), lambda i: (i, 0)),
    ],
    out_specs=pl.BlockSpec((D, D), lambda i: (0, 0)),
    out_shape=jax.ShapeDtypeStruct((D, D), jnp.float32),
)


def kernel(x, edge_index, batch, weight, w_ih, w_hh, b_ih, b_hh):
    src = edge_index[0].astype(jnp.int32)
    dst = edge_index[1].astype(jnp.int32)
    wih_t = w_ih.T
    whh_t = w_hh.T
    bih = b_ih.reshape(1, 3 * D)
    bhh = b_hh.reshape(1, 3 * D)
    zeros = jnp.zeros((N_PAD, D), jnp.float32)
    batch_b = jnp.broadcast_to(
        batch.astype(jnp.float32)[:, None], (N_NODES, D))

    sc_aggregate = _make_sc_aggregate()
    h = x
    m = _mm(x, weight[0])
    for i in range(NUM_LAYERS - 1):
        part = sc_aggregate(m, src, dst, zeros)
        h, m = _gru_mm(part, h, wih_t, whh_t, bih, bhh, weight[i + 1])
    part = sc_aggregate(m, src, dst, zeros)
    out_pad = _gru_pool(part, h, wih_t, whh_t, bih, bhh, batch_b)
    return out_pad[:N_GRAPHS]
